# baseline (speedup 1.0000x reference)
import functools

import jax
import jax.numpy as jnp
from jax import lax
from jax.experimental import pallas as pl
from jax.experimental.pallas import tpu as pltpu
from jax.experimental.pallas import tpu_sc as plsc

N = 50000
E = 800000
CI = 64
CM = 32
CO = 64

N_PAD = 50176
NP = N_PAD // 4
NR = N // 4
BLKP = 1568
NBLKP = NP // BLKP
CHUNK = 128
E_PAD = 819200
GRP = 16
NS = 16
RPT = N_PAD // NS
ZR = 224
ZDMA = RPT // ZR
CPT0 = 256

_f32 = jnp.float32


def _sc_apply_body(cpt0, cpt1, table, srcr, dstr, out, src_v, dst_v,
                   rows_a, rows_b, rows_c, zrow_v, acc_sh, gsem_a, gsem_b,
                   gsem_c, ssem_a, ssem_b, ssem_c):
    c = lax.axis_index("c")
    s = lax.axis_index("s")

    def _zfill(i, carry):
        zrow_v[i, pl.ds(0, 16)] = jnp.zeros((16,), _f32)
        zrow_v[i, pl.ds(16, 16)] = jnp.zeros((16,), _f32)
        return carry

    lax.fori_loop(0, ZR, _zfill, 0)

    def _zdma(k, carry):
        pltpu.sync_copy(zrow_v, acc_sh.at[pl.ds(s * RPT + k * ZR, ZR)])
        return carry

    lax.fori_loop(0, ZDMA, _zdma, 0)
    plsc.subcore_barrier()

    ng = jnp.where(c == 0, cpt0, cpt1) // GRP
    chunk0 = jnp.where(c == 0, s * cpt0, NS * cpt0 + s * cpt1)

    rows = (rows_a, rows_b, rows_c)
    gsems = (gsem_a, gsem_b, gsem_c)
    ssems = (ssem_a, ssem_b, ssem_c)

    def _group(g, carry):
        base = chunk0 + g * GRP
        pltpu.sync_copy(srcr.at[pl.ds(base, GRP)], src_v)
        pltpu.sync_copy(dstr.at[pl.ds(base, GRP)], dst_v)
        gd = [None] * GRP
        sd = [None] * GRP
        gd[0] = pltpu.async_copy(table.at[src_v.at[0]], rows[0], gsems[0])
        for j in range(GRP):
            if j + 1 < GRP:
                if j >= 2:
                    sd[j - 2].wait()
                gd[j + 1] = pltpu.async_copy(table.at[src_v.at[j + 1]],
                                             rows[(j + 1) % 3],
                                             gsems[(j + 1) % 3])
            gd[j].wait()
            sd[j] = pltpu.async_copy(rows[j % 3], acc_sh.at[dst_v.at[j]],
                                     ssems[j % 3], add=True)
        for j in range(GRP - 3, GRP):
            sd[j].wait()
        return carry

    lax.fori_loop(0, ng, _group, 0)
    plsc.subcore_barrier()

    pltpu.sync_copy(acc_sh.at[pl.ds(s * RPT, RPT)],
                    out.at[c, pl.ds(s * RPT, RPT)])


def _sc_apply(table, srcr, dstr, cpt0=None):
    n_chunks = srcr.shape[0]
    if cpt0 is None:
        cpt0 = n_chunks // 32
    cpt1 = n_chunks // NS - cpt0
    mesh = plsc.VectorSubcoreMesh(core_axis_name="c", subcore_axis_name="s")
    fn = pl.kernel(
        functools.partial(_sc_apply_body, cpt0, cpt1),
        out_type=jax.ShapeDtypeStruct((2, N_PAD, CM), _f32),
        mesh=mesh,
        scratch_types=[
            pltpu.VMEM((GRP, CHUNK), jnp.int32),
            pltpu.VMEM((GRP, CHUNK), jnp.int32),
            pltpu.VMEM((CHUNK, CM), _f32),
            pltpu.VMEM((CHUNK, CM), _f32),
            pltpu.VMEM((CHUNK, CM), _f32),
            pltpu.VMEM((ZR, CM), _f32),
            pltpu.VMEM_SHARED((N_PAD, CM), _f32),
            pltpu.SemaphoreType.DMA,
            pltpu.SemaphoreType.DMA,
            pltpu.SemaphoreType.DMA,
            pltpu.SemaphoreType.DMA,
            pltpu.SemaphoreType.DMA,
            pltpu.SemaphoreType.DMA,
        ],
        compiler_params=pltpu.CompilerParams(use_tc_tiling_on_sc=False),
    )
    return fn(table, srcr, dstr)


def _row_mask(i):
    rows = lax.broadcasted_iota(jnp.int32, (BLKP, 1), 0) + i * BLKP
    return rows < NR


def _bn_coeffs(s_ref, q_ref, g_ref, b_ref, rep=4):
    mean = s_ref[...] * (1.0 / N)
    var = q_ref[...] * (1.0 / N) - mean * mean
    inv = lax.rsqrt(var + 1e-3)
    scale = inv * g_ref[...]
    shift = b_ref[...] - mean * scale
    return (jnp.concatenate([scale] * rep, axis=1),
            jnp.concatenate([shift] * rep, axis=1))


_g1 = lambda i: (i, 0)
_g0 = lambda i: (0, 0)
_g2 = lambda i: (0, i, 0)


def _vspec(w):
    return pl.BlockSpec((1, w), _g0)


def _pk(w=128):
    return pl.BlockSpec((BLKP, w), _g1)


def _pk2(w=128):
    return pl.BlockSpec((2, BLKP, w), _g2)


def _tc(body, out_shape, in_specs, out_specs):
    return pl.pallas_call(
        body,
        grid=(NBLKP,),
        out_shape=out_shape,
        in_specs=in_specs,
        out_specs=out_specs,
    )


def _sh(r, w):
    return jax.ShapeDtypeStruct((r, w), _f32)


def _k0_body(x_ref, w1_ref, wr_ref, o_ref):
    i = pl.program_id(0)
    xb = jnp.where(_row_mask(i), x_ref[...], 0.0)
    o_ref[0] = jnp.dot(xb, w1_ref[...], preferred_element_type=_f32)
    o_ref[1] = jnp.dot(xb, wr_ref[...], preferred_element_type=_f32)


def _k0(xp, w1k, wrk):
    return _tc(
        _k0_body,
        jax.ShapeDtypeStruct((2, NP, 128), _f32),
        [_pk(256), pl.BlockSpec((256, 128), _g0),
         pl.BlockSpec((256, 128), _g0)],
        _pk2(),
    )(xp, w1k, wrk)


def _stats_pair_body(p_ref, f_ref, s0, q0, s1, q1):
    i = pl.program_id(0)

    @pl.when(i == 0)
    def _():
        for r in (s0, q0, s1, q1):
            r[...] = jnp.zeros_like(r)

    f = f_ref[...]
    h0 = p_ref[0]
    h1 = p_ref[1]
    s0[...] += jnp.dot(jnp.sum(h0, 0, keepdims=True), f,
                       preferred_element_type=_f32)
    q0[...] += jnp.dot(jnp.sum(h0 * h0, 0, keepdims=True), f,
                       preferred_element_type=_f32)
    s1[...] += jnp.dot(jnp.sum(h1, 0, keepdims=True), f,
                       preferred_element_type=_f32)
    q1[...] += jnp.dot(jnp.sum(h1 * h1, 0, keepdims=True), f,
                       preferred_element_type=_f32)


def _stats_pair(p4, f32m):
    sh = _sh(1, CM)
    return _tc(
        _stats_pair_body,
        (sh, sh, sh, sh),
        [_pk2(), pl.BlockSpec((128, CM), _g0)],
        tuple(_vspec(CM) for _ in range(4)),
    )(p4, f32m)


def _stats_sum_body(p_ref, f_ref, s0, q0):
    i = pl.program_id(0)

    @pl.when(i == 0)
    def _():
        s0[...] = jnp.zeros_like(s0)
        q0[...] = jnp.zeros_like(q0)

    f = f_ref[...]
    h = p_ref[0] + p_ref[1]
    s0[...] += jnp.dot(jnp.sum(h, 0, keepdims=True), f,
                       preferred_element_type=_f32)
    q0[...] += jnp.dot(jnp.sum(h * h, 0, keepdims=True), f,
                       preferred_element_type=_f32)


def _stats_sum(p4, f32m):
    sh = _sh(1, CM)
    return _tc(
        _stats_sum_body,
        (sh, sh),
        [_pk2(), pl.BlockSpec((128, CM), _g0)],
        (_vspec(CM), _vspec(CM)),
    )(p4, f32m)


def _k2_body(p_ref, s0, q0, s1, q1, g1r, b1r, grr, brr, w2r, u2_o, r_o):
    i = pl.program_id(0)
    m = _row_mask(i)
    sc0, sh0 = _bn_coeffs(s0, q0, g1r, b1r)
    sc1, sh1 = _bn_coeffs(s1, q1, grr, brr)
    d1 = jnp.where(m, jnp.maximum(p_ref[0] * sc0 + sh0, 0.0), 0.0)
    r = jnp.where(m, jnp.maximum(p_ref[1] * sc1 + sh1, 0.0), 0.0)
    u2_o[...] = jnp.dot(d1, w2r[...], preferred_element_type=_f32)
    r_o[...] = r


def _k2(p4, st, g1, b1, gr, br, w2k):
    return _tc(
        _k2_body,
        (_sh(NP, 128), _sh(NP, 128)),
        [_pk2()] + [_vspec(CM)] * 8 + [pl.BlockSpec((128, 128), _g0)],
        (_pk(), _pk()),
    )(p4, *st, g1, b1, gr, br, w2k)


def _k4_body(q_ref, s0, q0, gr, br, r_ref, w_ref, d3s_o, u_o):
    i = pl.program_id(0)
    sc, sh = _bn_coeffs(s0, q0, gr, br)
    h = q_ref[0] + q_ref[1]
    d2 = jnp.where(_row_mask(i), jnp.maximum(h * sc + sh, 0.0), 0.0)
    d3s = d2 + r_ref[...]
    d3s_o[...] = d3s
    u_o[...] = jnp.dot(d3s, w_ref[...], preferred_element_type=_f32)


def _k4(q4, st, g, b, rp, wk):
    return _tc(
        _k4_body,
        (_sh(NP, 128), _sh(NP, 128)),
        [_pk2()] + [_vspec(CM)] * 4
        + [_pk(), pl.BlockSpec((128, 128), _g0)],
        (_pk(), _pk()),
    )(q4, *st, g, b, rp, wk)


def _k6_body(p_ref, s0, q0, gr, br, w_ref, u_o):
    i = pl.program_id(0)
    sc, sh = _bn_coeffs(s0, q0, gr, br)
    h = p_ref[0] + p_ref[1]
    t = jnp.where(_row_mask(i), jnp.maximum(h * sc + sh, 0.0), 0.0)
    u_o[...] = jnp.dot(t, w_ref[...], preferred_element_type=_f32)


def _k6(p4, st, g, b, wk):
    return _tc(
        _k6_body,
        _sh(NP, 128),
        [_pk2()] + [_vspec(CM)] * 4 + [pl.BlockSpec((128, 128), _g0)],
        _pk(),
    )(p4, *st, g, b, wk)


def _k8_body(p_ref, s0, q0, gr, br, d3s_ref, f_ref, z_o, zs_o, zq_o):
    i = pl.program_id(0)

    @pl.when(i == 0)
    def _():
        zs_o[...] = jnp.zeros_like(zs_o)
        zq_o[...] = jnp.zeros_like(zq_o)

    sc, sh = _bn_coeffs(s0, q0, gr, br)
    h = p_ref[0] + p_ref[1]
    t = jnp.where(_row_mask(i), jnp.maximum(h * sc + sh, 0.0), 0.0)
    z = d3s_ref[...] + t
    z_o[...] = z
    f = f_ref[...]
    zs_o[...] += jnp.dot(jnp.sum(z, 0, keepdims=True), f,
                         preferred_element_type=_f32)
    zq_o[...] += jnp.dot(jnp.sum(z * z, 0, keepdims=True), f,
                         preferred_element_type=_f32)


def _k8(p4, st, g, b, d3sp, f32m):
    return _tc(
        _k8_body,
        (_sh(NP, 128), _sh(1, CM), _sh(1, CM)),
        [_pk2()] + [_vspec(CM)] * 4
        + [_pk(), pl.BlockSpec((128, CM), _g0)],
        (_pk(), _vspec(CM), _vspec(CM)),
    )(p4, *st, g, b, d3sp, f32m)


def _k9_body(z_ref, s0, q0, gr, br, w_ref, u_o):
    i = pl.program_id(0)
    sc, sh = _bn_coeffs(s0, q0, gr, br)
    mid = jnp.where(_row_mask(i), z_ref[...] * sc + sh, 0.0)
    u_o[...] = jnp.dot(mid, w_ref[...], preferred_element_type=_f32)


def _k9(zp, st, g, b, wk):
    return _tc(
        _k9_body,
        _sh(NP, 128),
        [_pk()] + [_vspec(CM)] * 4 + [pl.BlockSpec((128, 128), _g0)],
        _pk(),
    )(zp, *st, g, b, wk)


def _k11_body(p_ref, s0, q0, gr, br, e_o):
    i = pl.program_id(0)
    sc, sh = _bn_coeffs(s0, q0, gr, br)
    h = p_ref[0] + p_ref[1]
    e_o[...] = jnp.where(_row_mask(i), jnp.maximum(h * sc + sh, 0.0), 0.0)


def _k11(p4, st, g, b):
    return _tc(
        _k11_body,
        _sh(NP, 128),
        [_pk2()] + [_vspec(CM)] * 4,
        _pk(),
    )(p4, *st, g, b)


def _k12_body(p_ref, w_ref, f_ref, a_o, as_o, aq_o):
    i = pl.program_id(0)

    @pl.when(i == 0)
    def _():
        as_o[...] = jnp.zeros_like(as_o)
        aq_o[...] = jnp.zeros_like(aq_o)

    h = p_ref[0] + p_ref[1]
    a = jnp.dot(h, w_ref[...], preferred_element_type=_f32)
    a_o[...] = a
    f = f_ref[...]
    as_o[...] += jnp.dot(jnp.sum(a, 0, keepdims=True), f,
                         preferred_element_type=_f32)
    aq_o[...] += jnp.dot(jnp.sum(a * a, 0, keepdims=True), f,
                         preferred_element_type=_f32)


def _k12(p4, w4k, f64m):
    return _tc(
        _k12_body,
        (_sh(NP, 256), _sh(1, CO), _sh(1, CO)),
        [_pk2(), pl.BlockSpec((128, 256), _g0),
         pl.BlockSpec((256, CO), _g0)],
        (_pk(256), _vspec(CO), _vspec(CO)),
    )(p4, w4k, f64m)


def _k13_body(a_ref, s0, q0, gr, br, x_ref, f_ref, y_o, ys_o, yq_o):
    i = pl.program_id(0)

    @pl.when(i == 0)
    def _():
        ys_o[...] = jnp.zeros_like(ys_o)
        yq_o[...] = jnp.zeros_like(yq_o)

    sc, sh = _bn_coeffs(s0, q0, gr, br)
    t = jnp.maximum(a_ref[...] * sc + sh, 0.0)
    y = jnp.where(_row_mask(i), t + x_ref[...], 0.0)
    y_o[...] = y
    f = f_ref[...]
    ys_o[...] += jnp.dot(jnp.sum(y, 0, keepdims=True), f,
                         preferred_element_type=_f32)
    yq_o[...] += jnp.dot(jnp.sum(y * y, 0, keepdims=True), f,
                         preferred_element_type=_f32)


def _k13(a4p, st, g, b, xp, f64m):
    return _tc(
        _k13_body,
        (_sh(NR, 256), _sh(1, CO), _sh(1, CO)),
        [_pk(256)] + [_vspec(CO)] * 4
        + [_pk(256), pl.BlockSpec((256, CO), _g0)],
        (_pk(256), _vspec(CO), _vspec(CO)),
    )(a4p, *st, g, b, xp, f64m)


def _k14_body(y_ref, s0, q0, gr, br, o_ref):
    sc, sh = _bn_coeffs(s0, q0, gr, br)
    o_ref[...] = y_ref[...] * sc + sh


def _k14(yp, st, g, b):
    return _tc(
        _k14_body,
        _sh(NR, 256),
        [_pk(256)] + [_vspec(CO)] * 4,
        _pk(256),
    )(yp, *st, g, b)


def _prep_edges(ei):
    pad = jnp.full((E_PAD - E,), N, jnp.int32)
    src = jnp.concatenate([ei[0], pad])
    dst = jnp.concatenate([ei[1], pad])
    return src.reshape(-1, CHUNK), dst.reshape(-1, CHUNK), src, dst


def kernel(x, edge_index, edge_index_2d, W1, Wres, W2, W2d1, W2d2, W3, W4,
           g1, b1, gres, bres, g2, b2, g2d1, b2d1, g2d2, b2d2, g3, b3,
           g4, b4, gmid, bmid, gout, bout):
    v = lambda a: a.reshape(1, -1).astype(_f32)
    kr4 = lambda w: jnp.kron(jnp.eye(4, dtype=_f32), w)
    f32m = jnp.tile(jnp.eye(CM, dtype=_f32), (4, 1))
    f64m = jnp.tile(jnp.eye(CO, dtype=_f32), (4, 1))
    pk = lambda a: a.reshape(2, NP, 128)
    un = lambda a: a.reshape(-1, CM)

    src3r, dst3r, src3, dst3 = _prep_edges(edge_index)
    src2r, dst2r, _, _ = _prep_edges(edge_index_2d)
    src3p = jnp.concatenate([src3, src3 + N_PAD]).reshape(-1, CHUNK)
    dst3p = jnp.concatenate([dst3, dst3]).reshape(-1, CHUNK)

    xp = x.reshape(NR, 256)

    u = _k0(xp, kr4(W1), kr4(Wres))
    p = pk(_sc_apply(un(u), src3p, dst3p))
    st = _stats_pair(p, f32m)
    u2, r = _k2(p, st, v(g1), v(b1), v(gres), v(bres), kr4(W2))

    q = pk(_sc_apply(un(u2), src3r, dst3r, cpt0=CPT0))
    d3s, u2d1 = _k4(q, _stats_sum(q, f32m), v(g2), v(b2), r, kr4(W2d1))

    r3 = pk(_sc_apply(un(u2d1), src2r, dst2r, cpt0=CPT0))
    u2d2 = _k6(r3, _stats_sum(r3, f32m), v(g2d1), v(b2d1), kr4(W2d2))

    r4 = pk(_sc_apply(un(u2d2), src2r, dst2r, cpt0=CPT0))
    z, zs, zq = _k8(r4, _stats_sum(r4, f32m), v(g2d2), v(b2d2), d3s, f32m)

    u3 = _k9(z, (zs, zq), v(gmid), v(bmid), kr4(W3))

    r5 = pk(_sc_apply(un(u3), src3r, dst3r, cpt0=CPT0))
    e = _k11(r5, _stats_sum(r5, f32m), v(g3), v(b3))

    r6 = pk(_sc_apply(un(e), src3r, dst3r, cpt0=CPT0))
    a4, a4s, a4q = _k12(r6, kr4(W4), f64m)
    y, ys, yq = _k13(a4, (a4s, a4q), v(g4), v(b4), xp, f64m)
    return _k14(y, (ys, yq), v(gout), v(bout)).reshape(N, CO)


# speedup vs baseline: 1.0172x; 1.0172x over previous
import functools

import jax
import jax.numpy as jnp
from jax import lax
from jax.experimental import pallas as pl
from jax.experimental.pallas import tpu as pltpu
from jax.experimental.pallas import tpu_sc as plsc

N = 50000
E = 800000
CI = 64
CM = 32
CO = 64

N_PAD = 50176
NP = N_PAD // 4
NR = N // 4
BLKP = 1568
NBLKP = NP // BLKP
CHUNK = 128
E_PAD = 819200
GRP = 16
NS = 16
RPT = N_PAD // NS
ZR = 64
ZDMA = RPT // ZR
CPT0 = 288

_f32 = jnp.float32


def _sc_apply_body(cpt0, cpt1, table, srcr, dstr, out, src_v, dst_v,
                   rows_a, rows_b, rows_c, rows_d, zrow_v, acc_sh, gsem_a,
                   gsem_b, gsem_c, gsem_d, ssem_a, ssem_b, ssem_c, ssem_d):
    c = lax.axis_index("c")
    s = lax.axis_index("s")

    def _zfill(i, carry):
        zrow_v[i, pl.ds(0, 16)] = jnp.zeros((16,), _f32)
        zrow_v[i, pl.ds(16, 16)] = jnp.zeros((16,), _f32)
        return carry

    lax.fori_loop(0, ZR, _zfill, 0)

    def _zdma(k, carry):
        pltpu.sync_copy(zrow_v, acc_sh.at[pl.ds(s * RPT + k * ZR, ZR)])
        return carry

    lax.fori_loop(0, ZDMA, _zdma, 0)
    plsc.subcore_barrier()

    ng = jnp.where(c == 0, cpt0, cpt1) // GRP
    chunk0 = jnp.where(c == 0, s * cpt0, NS * cpt0 + s * cpt1)

    rows = (rows_a, rows_b, rows_c, rows_d)
    gsems = (gsem_a, gsem_b, gsem_c, gsem_d)
    ssems = (ssem_a, ssem_b, ssem_c, ssem_d)

    def _group(g, carry):
        base = chunk0 + g * GRP
        pltpu.sync_copy(srcr.at[pl.ds(base, GRP)], src_v)
        pltpu.sync_copy(dstr.at[pl.ds(base, GRP)], dst_v)
        gd = [None] * GRP
        sd = [None] * GRP
        gd[0] = pltpu.async_copy(table.at[src_v.at[0]], rows[0], gsems[0])
        for j in range(GRP):
            if j + 1 < GRP:
                if j >= 3:
                    sd[j - 3].wait()
                gd[j + 1] = pltpu.async_copy(table.at[src_v.at[j + 1]],
                                             rows[(j + 1) % 4],
                                             gsems[(j + 1) % 4])
            gd[j].wait()
            sd[j] = pltpu.async_copy(rows[j % 4], acc_sh.at[dst_v.at[j]],
                                     ssems[j % 4], add=True)
        for j in range(GRP - 4, GRP):
            sd[j].wait()
        return carry

    lax.fori_loop(0, ng, _group, 0)
    plsc.subcore_barrier()

    pltpu.sync_copy(acc_sh.at[pl.ds(s * RPT, RPT)],
                    out.at[c, pl.ds(s * RPT, RPT)])


def _sc_apply(table, srcr, dstr, cpt0=None):
    n_chunks = srcr.shape[0]
    if cpt0 is None:
        cpt0 = n_chunks // 32
    cpt1 = n_chunks // NS - cpt0
    mesh = plsc.VectorSubcoreMesh(core_axis_name="c", subcore_axis_name="s")
    fn = pl.kernel(
        functools.partial(_sc_apply_body, cpt0, cpt1),
        out_type=jax.ShapeDtypeStruct((2, N_PAD, CM), _f32),
        mesh=mesh,
        scratch_types=[
            pltpu.VMEM((GRP, CHUNK), jnp.int32),
            pltpu.VMEM((GRP, CHUNK), jnp.int32),
            pltpu.VMEM((CHUNK, CM), _f32),
            pltpu.VMEM((CHUNK, CM), _f32),
            pltpu.VMEM((CHUNK, CM), _f32),
            pltpu.VMEM((CHUNK, CM), _f32),
            pltpu.VMEM((ZR, CM), _f32),
            pltpu.VMEM_SHARED((N_PAD, CM), _f32),
            pltpu.SemaphoreType.DMA,
            pltpu.SemaphoreType.DMA,
            pltpu.SemaphoreType.DMA,
            pltpu.SemaphoreType.DMA,
            pltpu.SemaphoreType.DMA,
            pltpu.SemaphoreType.DMA,
            pltpu.SemaphoreType.DMA,
            pltpu.SemaphoreType.DMA,
        ],
        compiler_params=pltpu.CompilerParams(use_tc_tiling_on_sc=False),
    )
    return fn(table, srcr, dstr)


def _row_mask(i):
    rows = lax.broadcasted_iota(jnp.int32, (BLKP, 1), 0) + i * BLKP
    return rows < NR


def _bn_coeffs(s_ref, q_ref, g_ref, b_ref, rep=4):
    mean = s_ref[...] * (1.0 / N)
    var = q_ref[...] * (1.0 / N) - mean * mean
    inv = lax.rsqrt(var + 1e-3)
    scale = inv * g_ref[...]
    shift = b_ref[...] - mean * scale
    return (jnp.concatenate([scale] * rep, axis=1),
            jnp.concatenate([shift] * rep, axis=1))


_g1 = lambda i: (i, 0)
_g0 = lambda i: (0, 0)
_g2 = lambda i: (0, i, 0)


def _vspec(w):
    return pl.BlockSpec((1, w), _g0)


def _pk(w=128):
    return pl.BlockSpec((BLKP, w), _g1)


def _pk2(w=128):
    return pl.BlockSpec((2, BLKP, w), _g2)


def _tc(body, out_shape, in_specs, out_specs):
    return pl.pallas_call(
        body,
        grid=(NBLKP,),
        out_shape=out_shape,
        in_specs=in_specs,
        out_specs=out_specs,
    )


def _sh(r, w):
    return jax.ShapeDtypeStruct((r, w), _f32)


def _k0_body(x_ref, w1_ref, wr_ref, o_ref):
    i = pl.program_id(0)
    xb = jnp.where(_row_mask(i), x_ref[...], 0.0)
    o_ref[0] = jnp.dot(xb, w1_ref[...], preferred_element_type=_f32)
    o_ref[1] = jnp.dot(xb, wr_ref[...], preferred_element_type=_f32)


def _k0(xp, w1k, wrk):
    return _tc(
        _k0_body,
        jax.ShapeDtypeStruct((2, NP, 128), _f32),
        [_pk(256), pl.BlockSpec((256, 128), _g0),
         pl.BlockSpec((256, 128), _g0)],
        _pk2(),
    )(xp, w1k, wrk)


def _stats_pair_body(p_ref, f_ref, s0, q0, s1, q1):
    i = pl.program_id(0)

    @pl.when(i == 0)
    def _():
        for r in (s0, q0, s1, q1):
            r[...] = jnp.zeros_like(r)

    f = f_ref[...]
    h0 = p_ref[0]
    h1 = p_ref[1]
    s0[...] += jnp.dot(jnp.sum(h0, 0, keepdims=True), f,
                       preferred_element_type=_f32)
    q0[...] += jnp.dot(jnp.sum(h0 * h0, 0, keepdims=True), f,
                       preferred_element_type=_f32)
    s1[...] += jnp.dot(jnp.sum(h1, 0, keepdims=True), f,
                       preferred_element_type=_f32)
    q1[...] += jnp.dot(jnp.sum(h1 * h1, 0, keepdims=True), f,
                       preferred_element_type=_f32)


def _stats_pair(p4, f32m):
    sh = _sh(1, CM)
    return _tc(
        _stats_pair_body,
        (sh, sh, sh, sh),
        [_pk2(), pl.BlockSpec((128, CM), _g0)],
        tuple(_vspec(CM) for _ in range(4)),
    )(p4, f32m)


def _stats_sum_body(p_ref, f_ref, s0, q0):
    i = pl.program_id(0)

    @pl.when(i == 0)
    def _():
        s0[...] = jnp.zeros_like(s0)
        q0[...] = jnp.zeros_like(q0)

    f = f_ref[...]
    h = p_ref[0] + p_ref[1]
    s0[...] += jnp.dot(jnp.sum(h, 0, keepdims=True), f,
                       preferred_element_type=_f32)
    q0[...] += jnp.dot(jnp.sum(h * h, 0, keepdims=True), f,
                       preferred_element_type=_f32)


def _stats_sum(p4, f32m):
    sh = _sh(1, CM)
    return _tc(
        _stats_sum_body,
        (sh, sh),
        [_pk2(), pl.BlockSpec((128, CM), _g0)],
        (_vspec(CM), _vspec(CM)),
    )(p4, f32m)


def _k2_body(p_ref, s0, q0, s1, q1, g1r, b1r, grr, brr, w2r, u2_o, r_o):
    i = pl.program_id(0)
    m = _row_mask(i)
    sc0, sh0 = _bn_coeffs(s0, q0, g1r, b1r)
    sc1, sh1 = _bn_coeffs(s1, q1, grr, brr)
    d1 = jnp.where(m, jnp.maximum(p_ref[0] * sc0 + sh0, 0.0), 0.0)
    r = jnp.where(m, jnp.maximum(p_ref[1] * sc1 + sh1, 0.0), 0.0)
    u2_o[...] = jnp.dot(d1, w2r[...], preferred_element_type=_f32)
    r_o[...] = r


def _k2(p4, st, g1, b1, gr, br, w2k):
    return _tc(
        _k2_body,
        (_sh(NP, 128), _sh(NP, 128)),
        [_pk2()] + [_vspec(CM)] * 8 + [pl.BlockSpec((128, 128), _g0)],
        (_pk(), _pk()),
    )(p4, *st, g1, b1, gr, br, w2k)


def _k4_body(q_ref, s0, q0, gr, br, r_ref, w_ref, d3s_o, u_o):
    i = pl.program_id(0)
    sc, sh = _bn_coeffs(s0, q0, gr, br)
    h = q_ref[0] + q_ref[1]
    d2 = jnp.where(_row_mask(i), jnp.maximum(h * sc + sh, 0.0), 0.0)
    d3s = d2 + r_ref[...]
    d3s_o[...] = d3s
    u_o[...] = jnp.dot(d3s, w_ref[...], preferred_element_type=_f32)


def _k4(q4, st, g, b, rp, wk):
    return _tc(
        _k4_body,
        (_sh(NP, 128), _sh(NP, 128)),
        [_pk2()] + [_vspec(CM)] * 4
        + [_pk(), pl.BlockSpec((128, 128), _g0)],
        (_pk(), _pk()),
    )(q4, *st, g, b, rp, wk)


def _k6_body(p_ref, s0, q0, gr, br, w_ref, u_o):
    i = pl.program_id(0)
    sc, sh = _bn_coeffs(s0, q0, gr, br)
    h = p_ref[0] + p_ref[1]
    t = jnp.where(_row_mask(i), jnp.maximum(h * sc + sh, 0.0), 0.0)
    u_o[...] = jnp.dot(t, w_ref[...], preferred_element_type=_f32)


def _k6(p4, st, g, b, wk):
    return _tc(
        _k6_body,
        _sh(NP, 128),
        [_pk2()] + [_vspec(CM)] * 4 + [pl.BlockSpec((128, 128), _g0)],
        _pk(),
    )(p4, *st, g, b, wk)


def _k8_body(p_ref, s0, q0, gr, br, d3s_ref, f_ref, z_o, zs_o, zq_o):
    i = pl.program_id(0)

    @pl.when(i == 0)
    def _():
        zs_o[...] = jnp.zeros_like(zs_o)
        zq_o[...] = jnp.zeros_like(zq_o)

    sc, sh = _bn_coeffs(s0, q0, gr, br)
    h = p_ref[0] + p_ref[1]
    t = jnp.where(_row_mask(i), jnp.maximum(h * sc + sh, 0.0), 0.0)
    z = d3s_ref[...] + t
    z_o[...] = z
    f = f_ref[...]
    zs_o[...] += jnp.dot(jnp.sum(z, 0, keepdims=True), f,
                         preferred_element_type=_f32)
    zq_o[...] += jnp.dot(jnp.sum(z * z, 0, keepdims=True), f,
                         preferred_element_type=_f32)


def _k8(p4, st, g, b, d3sp, f32m):
    return _tc(
        _k8_body,
        (_sh(NP, 128), _sh(1, CM), _sh(1, CM)),
        [_pk2()] + [_vspec(CM)] * 4
        + [_pk(), pl.BlockSpec((128, CM), _g0)],
        (_pk(), _vspec(CM), _vspec(CM)),
    )(p4, *st, g, b, d3sp, f32m)


def _k9_body(z_ref, s0, q0, gr, br, w_ref, u_o):
    i = pl.program_id(0)
    sc, sh = _bn_coeffs(s0, q0, gr, br)
    mid = jnp.where(_row_mask(i), z_ref[...] * sc + sh, 0.0)
    u_o[...] = jnp.dot(mid, w_ref[...], preferred_element_type=_f32)


def _k9(zp, st, g, b, wk):
    return _tc(
        _k9_body,
        _sh(NP, 128),
        [_pk()] + [_vspec(CM)] * 4 + [pl.BlockSpec((128, 128), _g0)],
        _pk(),
    )(zp, *st, g, b, wk)


def _k11_body(p_ref, s0, q0, gr, br, e_o):
    i = pl.program_id(0)
    sc, sh = _bn_coeffs(s0, q0, gr, br)
    h = p_ref[0] + p_ref[1]
    e_o[...] = jnp.where(_row_mask(i), jnp.maximum(h * sc + sh, 0.0), 0.0)


def _k11(p4, st, g, b):
    return _tc(
        _k11_body,
        _sh(NP, 128),
        [_pk2()] + [_vspec(CM)] * 4,
        _pk(),
    )(p4, *st, g, b)


def _k12_body(p_ref, w_ref, f_ref, a_o, as_o, aq_o):
    i = pl.program_id(0)

    @pl.when(i == 0)
    def _():
        as_o[...] = jnp.zeros_like(as_o)
        aq_o[...] = jnp.zeros_like(aq_o)

    h = p_ref[0] + p_ref[1]
    a = jnp.dot(h, w_ref[...], preferred_element_type=_f32)
    a_o[...] = a
    f = f_ref[...]
    as_o[...] += jnp.dot(jnp.sum(a, 0, keepdims=True), f,
                         preferred_element_type=_f32)
    aq_o[...] += jnp.dot(jnp.sum(a * a, 0, keepdims=True), f,
                         preferred_element_type=_f32)


def _k12(p4, w4k, f64m):
    return _tc(
        _k12_body,
        (_sh(NP, 256), _sh(1, CO), _sh(1, CO)),
        [_pk2(), pl.BlockSpec((128, 256), _g0),
         pl.BlockSpec((256, CO), _g0)],
        (_pk(256), _vspec(CO), _vspec(CO)),
    )(p4, w4k, f64m)


def _k13_body(a_ref, s0, q0, gr, br, x_ref, f_ref, y_o, ys_o, yq_o):
    i = pl.program_id(0)

    @pl.when(i == 0)
    def _():
        ys_o[...] = jnp.zeros_like(ys_o)
        yq_o[...] = jnp.zeros_like(yq_o)

    sc, sh = _bn_coeffs(s0, q0, gr, br)
    t = jnp.maximum(a_ref[...] * sc + sh, 0.0)
    y = jnp.where(_row_mask(i), t + x_ref[...], 0.0)
    y_o[...] = y
    f = f_ref[...]
    ys_o[...] += jnp.dot(jnp.sum(y, 0, keepdims=True), f,
                         preferred_element_type=_f32)
    yq_o[...] += jnp.dot(jnp.sum(y * y, 0, keepdims=True), f,
                         preferred_element_type=_f32)


def _k13(a4p, st, g, b, xp, f64m):
    return _tc(
        _k13_body,
        (_sh(NR, 256), _sh(1, CO), _sh(1, CO)),
        [_pk(256)] + [_vspec(CO)] * 4
        + [_pk(256), pl.BlockSpec((256, CO), _g0)],
        (_pk(256), _vspec(CO), _vspec(CO)),
    )(a4p, *st, g, b, xp, f64m)


def _k14_body(y_ref, s0, q0, gr, br, o_ref):
    sc, sh = _bn_coeffs(s0, q0, gr, br)
    o_ref[...] = y_ref[...] * sc + sh


def _k14(yp, st, g, b):
    return _tc(
        _k14_body,
        _sh(NR, 256),
        [_pk(256)] + [_vspec(CO)] * 4,
        _pk(256),
    )(yp, *st, g, b)


def _prep_edges(ei):
    pad = jnp.full((E_PAD - E,), N, jnp.int32)
    src = jnp.concatenate([ei[0], pad])
    dst = jnp.concatenate([ei[1], pad])
    return src.reshape(-1, CHUNK), dst.reshape(-1, CHUNK), src, dst


def kernel(x, edge_index, edge_index_2d, W1, Wres, W2, W2d1, W2d2, W3, W4,
           g1, b1, gres, bres, g2, b2, g2d1, b2d1, g2d2, b2d2, g3, b3,
           g4, b4, gmid, bmid, gout, bout):
    v = lambda a: a.reshape(1, -1).astype(_f32)
    kr4 = lambda w: jnp.kron(jnp.eye(4, dtype=_f32), w)
    f32m = jnp.tile(jnp.eye(CM, dtype=_f32), (4, 1))
    f64m = jnp.tile(jnp.eye(CO, dtype=_f32), (4, 1))
    pk = lambda a: a.reshape(2, NP, 128)
    un = lambda a: a.reshape(-1, CM)

    src3r, dst3r, src3, dst3 = _prep_edges(edge_index)
    src2r, dst2r, _, _ = _prep_edges(edge_index_2d)
    src3p = jnp.concatenate([src3, src3 + N_PAD]).reshape(-1, CHUNK)
    dst3p = jnp.concatenate([dst3, dst3]).reshape(-1, CHUNK)

    xp = x.reshape(NR, 256)

    u = _k0(xp, kr4(W1), kr4(Wres))
    p = pk(_sc_apply(un(u), src3p, dst3p))
    st = _stats_pair(p, f32m)
    u2, r = _k2(p, st, v(g1), v(b1), v(gres), v(bres), kr4(W2))

    q = pk(_sc_apply(un(u2), src3r, dst3r, cpt0=CPT0))
    d3s, u2d1 = _k4(q, _stats_sum(q, f32m), v(g2), v(b2), r, kr4(W2d1))

    r3 = pk(_sc_apply(un(u2d1), src2r, dst2r, cpt0=CPT0))
    u2d2 = _k6(r3, _stats_sum(r3, f32m), v(g2d1), v(b2d1), kr4(W2d2))

    r4 = pk(_sc_apply(un(u2d2), src2r, dst2r, cpt0=CPT0))
    z, zs, zq = _k8(r4, _stats_sum(r4, f32m), v(g2d2), v(b2d2), d3s, f32m)

    u3 = _k9(z, (zs, zq), v(gmid), v(bmid), kr4(W3))

    r5 = pk(_sc_apply(un(u3), src3r, dst3r, cpt0=CPT0))
    e = _k11(r5, _stats_sum(r5, f32m), v(g3), v(b3))

    r6 = pk(_sc_apply(un(e), src3r, dst3r, cpt0=CPT0))
    a4, a4s, a4q = _k12(r6, kr4(W4), f64m)
    y, ys, yq = _k13(a4, (a4s, a4q), v(g4), v(b4), xp, f64m)
    return _k14(y, (ys, yq), v(gout), v(bout)).reshape(N, CO)


# speedup vs baseline: 1.0373x; 1.0198x over previous
import functools

import jax
import jax.numpy as jnp
from jax import lax
from jax.experimental import pallas as pl
from jax.experimental.pallas import tpu as pltpu
from jax.experimental.pallas import tpu_sc as plsc

N = 50000
E = 800000
CI = 64
CM = 32
CO = 64

N_PAD = 50176
NP = N_PAD // 4
NR = N // 4
BLKP = 1568
NBLKP = NP // BLKP
CHUNK = 128
E_PAD = 819200
GRP = 16
NS = 16
RPT = N_PAD // NS
ZR = 224
ZDMA = RPT // ZR
CPT0 = 304

_f32 = jnp.float32


def _sc_apply_body(cpt0, cpt1, table, srcr, dstr, out, src_v, dst_v,
                   rows_a, rows_b, rows_c, zrow_v, acc_sh, gsem_a, gsem_b,
                   gsem_c, ssem_a, ssem_b, ssem_c):
    c = lax.axis_index("c")
    s = lax.axis_index("s")

    def _zfill(i, carry):
        zrow_v[i, pl.ds(0, 16)] = jnp.zeros((16,), _f32)
        zrow_v[i, pl.ds(16, 16)] = jnp.zeros((16,), _f32)
        return carry

    lax.fori_loop(0, ZR, _zfill, 0)

    def _zdma(k, carry):
        pltpu.sync_copy(zrow_v, acc_sh.at[pl.ds(s * RPT + k * ZR, ZR)])
        return carry

    lax.fori_loop(0, ZDMA, _zdma, 0)
    plsc.subcore_barrier()

    ng = jnp.where(c == 0, cpt0, cpt1) // GRP
    chunk0 = jnp.where(c == 0, s * cpt0, NS * cpt0 + s * cpt1)

    rows = (rows_a, rows_b, rows_c)
    gsems = (gsem_a, gsem_b, gsem_c)
    ssems = (ssem_a, ssem_b, ssem_c)

    def _group(g, carry):
        base = chunk0 + g * GRP
        pltpu.sync_copy(srcr.at[pl.ds(base, GRP)], src_v)
        pltpu.sync_copy(dstr.at[pl.ds(base, GRP)], dst_v)
        gd = [None] * GRP
        sd = [None] * GRP
        gd[0] = pltpu.async_copy(table.at[src_v.at[0]], rows[0], gsems[0])
        for j in range(GRP):
            if j + 1 < GRP:
                if j >= 2:
                    sd[j - 2].wait()
                gd[j + 1] = pltpu.async_copy(table.at[src_v.at[j + 1]],
                                             rows[(j + 1) % 3],
                                             gsems[(j + 1) % 3])
            gd[j].wait()
            sd[j] = pltpu.async_copy(rows[j % 3], acc_sh.at[dst_v.at[j]],
                                     ssems[j % 3], add=True)
        for j in range(GRP - 3, GRP):
            sd[j].wait()
        return carry

    lax.fori_loop(0, ng, _group, 0)
    plsc.subcore_barrier()

    pltpu.sync_copy(acc_sh.at[pl.ds(s * RPT, RPT)],
                    out.at[c, pl.ds(s * RPT, RPT)])


def _sc_apply(table, srcr, dstr, cpt0=None):
    n_chunks = srcr.shape[0]
    if cpt0 is None:
        cpt0 = n_chunks // 32
    cpt1 = n_chunks // NS - cpt0
    mesh = plsc.VectorSubcoreMesh(core_axis_name="c", subcore_axis_name="s")
    fn = pl.kernel(
        functools.partial(_sc_apply_body, cpt0, cpt1),
        out_type=jax.ShapeDtypeStruct((2, N_PAD, CM), _f32),
        mesh=mesh,
        scratch_types=[
            pltpu.VMEM((GRP, CHUNK), jnp.int32),
            pltpu.VMEM((GRP, CHUNK), jnp.int32),
            pltpu.VMEM((CHUNK, CM), _f32),
            pltpu.VMEM((CHUNK, CM), _f32),
            pltpu.VMEM((CHUNK, CM), _f32),
            pltpu.VMEM((ZR, CM), _f32),
            pltpu.VMEM_SHARED((N_PAD, CM), _f32),
            pltpu.SemaphoreType.DMA,
            pltpu.SemaphoreType.DMA,
            pltpu.SemaphoreType.DMA,
            pltpu.SemaphoreType.DMA,
            pltpu.SemaphoreType.DMA,
            pltpu.SemaphoreType.DMA,
        ],
        compiler_params=pltpu.CompilerParams(use_tc_tiling_on_sc=False),
    )
    return fn(table, srcr, dstr)


def _row_mask(i):
    rows = lax.broadcasted_iota(jnp.int32, (BLKP, 1), 0) + i * BLKP
    return rows < NR


def _bn_coeffs(s_ref, q_ref, g_ref, b_ref, rep=4):
    mean = s_ref[...] * (1.0 / N)
    var = q_ref[...] * (1.0 / N) - mean * mean
    inv = lax.rsqrt(var + 1e-3)
    scale = inv * g_ref[...]
    shift = b_ref[...] - mean * scale
    return (jnp.concatenate([scale] * rep, axis=1),
            jnp.concatenate([shift] * rep, axis=1))


_g1 = lambda i: (i, 0)
_g0 = lambda i: (0, 0)
_g2 = lambda i: (0, i, 0)


def _vspec(w):
    return pl.BlockSpec((1, w), _g0)


def _pk(w=128):
    return pl.BlockSpec((BLKP, w), _g1)


def _pk2(w=128):
    return pl.BlockSpec((2, BLKP, w), _g2)


def _tc(body, out_shape, in_specs, out_specs):
    return pl.pallas_call(
        body,
        grid=(NBLKP,),
        out_shape=out_shape,
        in_specs=in_specs,
        out_specs=out_specs,
    )


def _sh(r, w):
    return jax.ShapeDtypeStruct((r, w), _f32)


def _k0_body(x_ref, w1_ref, wr_ref, o_ref):
    i = pl.program_id(0)
    xb = jnp.where(_row_mask(i), x_ref[...], 0.0)
    o_ref[0] = jnp.dot(xb, w1_ref[...], preferred_element_type=_f32)
    o_ref[1] = jnp.dot(xb, wr_ref[...], preferred_element_type=_f32)


def _k0(xp, w1k, wrk):
    return _tc(
        _k0_body,
        jax.ShapeDtypeStruct((2, NP, 128), _f32),
        [_pk(256), pl.BlockSpec((256, 128), _g0),
         pl.BlockSpec((256, 128), _g0)],
        _pk2(),
    )(xp, w1k, wrk)


def _stats_pair_body(p_ref, f_ref, s0, q0, s1, q1):
    i = pl.program_id(0)

    @pl.when(i == 0)
    def _():
        for r in (s0, q0, s1, q1):
            r[...] = jnp.zeros_like(r)

    f = f_ref[...]
    h0 = p_ref[0]
    h1 = p_ref[1]
    s0[...] += jnp.dot(jnp.sum(h0, 0, keepdims=True), f,
                       preferred_element_type=_f32)
    q0[...] += jnp.dot(jnp.sum(h0 * h0, 0, keepdims=True), f,
                       preferred_element_type=_f32)
    s1[...] += jnp.dot(jnp.sum(h1, 0, keepdims=True), f,
                       preferred_element_type=_f32)
    q1[...] += jnp.dot(jnp.sum(h1 * h1, 0, keepdims=True), f,
                       preferred_element_type=_f32)


def _stats_pair(p4, f32m):
    sh = _sh(1, CM)
    return _tc(
        _stats_pair_body,
        (sh, sh, sh, sh),
        [_pk2(), pl.BlockSpec((128, CM), _g0)],
        tuple(_vspec(CM) for _ in range(4)),
    )(p4, f32m)


def _stats_sum_body(p_ref, f_ref, s0, q0):
    i = pl.program_id(0)

    @pl.when(i == 0)
    def _():
        s0[...] = jnp.zeros_like(s0)
        q0[...] = jnp.zeros_like(q0)

    f = f_ref[...]
    h = p_ref[0] + p_ref[1]
    s0[...] += jnp.dot(jnp.sum(h, 0, keepdims=True), f,
                       preferred_element_type=_f32)
    q0[...] += jnp.dot(jnp.sum(h * h, 0, keepdims=True), f,
                       preferred_element_type=_f32)


def _stats_sum(p4, f32m):
    sh = _sh(1, CM)
    return _tc(
        _stats_sum_body,
        (sh, sh),
        [_pk2(), pl.BlockSpec((128, CM), _g0)],
        (_vspec(CM), _vspec(CM)),
    )(p4, f32m)


def _k2_body(p_ref, s0, q0, s1, q1, g1r, b1r, grr, brr, w2r, u2_o, r_o):
    i = pl.program_id(0)
    m = _row_mask(i)
    sc0, sh0 = _bn_coeffs(s0, q0, g1r, b1r)
    sc1, sh1 = _bn_coeffs(s1, q1, grr, brr)
    d1 = jnp.where(m, jnp.maximum(p_ref[0] * sc0 + sh0, 0.0), 0.0)
    r = jnp.where(m, jnp.maximum(p_ref[1] * sc1 + sh1, 0.0), 0.0)
    u2_o[...] = jnp.dot(d1, w2r[...], preferred_element_type=_f32)
    r_o[...] = r


def _k2(p4, st, g1, b1, gr, br, w2k):
    return _tc(
        _k2_body,
        (_sh(NP, 128), _sh(NP, 128)),
        [_pk2()] + [_vspec(CM)] * 8 + [pl.BlockSpec((128, 128), _g0)],
        (_pk(), _pk()),
    )(p4, *st, g1, b1, gr, br, w2k)


def _k4_body(q_ref, s0, q0, gr, br, r_ref, w_ref, d3s_o, u_o):
    i = pl.program_id(0)
    sc, sh = _bn_coeffs(s0, q0, gr, br)
    h = q_ref[0] + q_ref[1]
    d2 = jnp.where(_row_mask(i), jnp.maximum(h * sc + sh, 0.0), 0.0)
    d3s = d2 + r_ref[...]
    d3s_o[...] = d3s
    u_o[...] = jnp.dot(d3s, w_ref[...], preferred_element_type=_f32)


def _k4(q4, st, g, b, rp, wk):
    return _tc(
        _k4_body,
        (_sh(NP, 128), _sh(NP, 128)),
        [_pk2()] + [_vspec(CM)] * 4
        + [_pk(), pl.BlockSpec((128, 128), _g0)],
        (_pk(), _pk()),
    )(q4, *st, g, b, rp, wk)


def _k6_body(p_ref, s0, q0, gr, br, w_ref, u_o):
    i = pl.program_id(0)
    sc, sh = _bn_coeffs(s0, q0, gr, br)
    h = p_ref[0] + p_ref[1]
    t = jnp.where(_row_mask(i), jnp.maximum(h * sc + sh, 0.0), 0.0)
    u_o[...] = jnp.dot(t, w_ref[...], preferred_element_type=_f32)


def _k6(p4, st, g, b, wk):
    return _tc(
        _k6_body,
        _sh(NP, 128),
        [_pk2()] + [_vspec(CM)] * 4 + [pl.BlockSpec((128, 128), _g0)],
        _pk(),
    )(p4, *st, g, b, wk)


def _k8_body(p_ref, s0, q0, gr, br, d3s_ref, f_ref, z_o, zs_o, zq_o):
    i = pl.program_id(0)

    @pl.when(i == 0)
    def _():
        zs_o[...] = jnp.zeros_like(zs_o)
        zq_o[...] = jnp.zeros_like(zq_o)

    sc, sh = _bn_coeffs(s0, q0, gr, br)
    h = p_ref[0] + p_ref[1]
    t = jnp.where(_row_mask(i), jnp.maximum(h * sc + sh, 0.0), 0.0)
    z = d3s_ref[...] + t
    z_o[...] = z
    f = f_ref[...]
    zs_o[...] += jnp.dot(jnp.sum(z, 0, keepdims=True), f,
                         preferred_element_type=_f32)
    zq_o[...] += jnp.dot(jnp.sum(z * z, 0, keepdims=True), f,
                         preferred_element_type=_f32)


def _k8(p4, st, g, b, d3sp, f32m):
    return _tc(
        _k8_body,
        (_sh(NP, 128), _sh(1, CM), _sh(1, CM)),
        [_pk2()] + [_vspec(CM)] * 4
        + [_pk(), pl.BlockSpec((128, CM), _g0)],
        (_pk(), _vspec(CM), _vspec(CM)),
    )(p4, *st, g, b, d3sp, f32m)


def _k9_body(z_ref, s0, q0, gr, br, w_ref, u_o):
    i = pl.program_id(0)
    sc, sh = _bn_coeffs(s0, q0, gr, br)
    mid = jnp.where(_row_mask(i), z_ref[...] * sc + sh, 0.0)
    u_o[...] = jnp.dot(mid, w_ref[...], preferred_element_type=_f32)


def _k9(zp, st, g, b, wk):
    return _tc(
        _k9_body,
        _sh(NP, 128),
        [_pk()] + [_vspec(CM)] * 4 + [pl.BlockSpec((128, 128), _g0)],
        _pk(),
    )(zp, *st, g, b, wk)


def _k11_body(p_ref, s0, q0, gr, br, e_o):
    i = pl.program_id(0)
    sc, sh = _bn_coeffs(s0, q0, gr, br)
    h = p_ref[0] + p_ref[1]
    e_o[...] = jnp.where(_row_mask(i), jnp.maximum(h * sc + sh, 0.0), 0.0)


def _k11(p4, st, g, b):
    return _tc(
        _k11_body,
        _sh(NP, 128),
        [_pk2()] + [_vspec(CM)] * 4,
        _pk(),
    )(p4, *st, g, b)


def _k12_body(p_ref, w_ref, f_ref, a_o, as_o, aq_o):
    i = pl.program_id(0)

    @pl.when(i == 0)
    def _():
        as_o[...] = jnp.zeros_like(as_o)
        aq_o[...] = jnp.zeros_like(aq_o)

    h = p_ref[0] + p_ref[1]
    a = jnp.dot(h, w_ref[...], preferred_element_type=_f32)
    a_o[...] = a
    f = f_ref[...]
    as_o[...] += jnp.dot(jnp.sum(a, 0, keepdims=True), f,
                         preferred_element_type=_f32)
    aq_o[...] += jnp.dot(jnp.sum(a * a, 0, keepdims=True), f,
                         preferred_element_type=_f32)


def _k12(p4, w4k, f64m):
    return _tc(
        _k12_body,
        (_sh(NP, 256), _sh(1, CO), _sh(1, CO)),
        [_pk2(), pl.BlockSpec((128, 256), _g0),
         pl.BlockSpec((256, CO), _g0)],
        (_pk(256), _vspec(CO), _vspec(CO)),
    )(p4, w4k, f64m)


def _k13_body(a_ref, s0, q0, gr, br, x_ref, f_ref, y_o, ys_o, yq_o):
    i = pl.program_id(0)

    @pl.when(i == 0)
    def _():
        ys_o[...] = jnp.zeros_like(ys_o)
        yq_o[...] = jnp.zeros_like(yq_o)

    sc, sh = _bn_coeffs(s0, q0, gr, br)
    t = jnp.maximum(a_ref[...] * sc + sh, 0.0)
    y = jnp.where(_row_mask(i), t + x_ref[...], 0.0)
    y_o[...] = y
    f = f_ref[...]
    ys_o[...] += jnp.dot(jnp.sum(y, 0, keepdims=True), f,
                         preferred_element_type=_f32)
    yq_o[...] += jnp.dot(jnp.sum(y * y, 0, keepdims=True), f,
                         preferred_element_type=_f32)


def _k13(a4p, st, g, b, xp, f64m):
    return _tc(
        _k13_body,
        (_sh(NR, 256), _sh(1, CO), _sh(1, CO)),
        [_pk(256)] + [_vspec(CO)] * 4
        + [_pk(256), pl.BlockSpec((256, CO), _g0)],
        (_pk(256), _vspec(CO), _vspec(CO)),
    )(a4p, *st, g, b, xp, f64m)


def _k14_body(y_ref, s0, q0, gr, br, o_ref):
    sc, sh = _bn_coeffs(s0, q0, gr, br)
    o_ref[...] = y_ref[...] * sc + sh


def _k14(yp, st, g, b):
    return _tc(
        _k14_body,
        _sh(NR, 256),
        [_pk(256)] + [_vspec(CO)] * 4,
        _pk(256),
    )(yp, *st, g, b)


def _prep_edges(ei):
    pad = jnp.full((E_PAD - E,), N, jnp.int32)
    src = jnp.concatenate([ei[0], pad])
    dst = jnp.concatenate([ei[1], pad])
    return src.reshape(-1, CHUNK), dst.reshape(-1, CHUNK), src, dst


def kernel(x, edge_index, edge_index_2d, W1, Wres, W2, W2d1, W2d2, W3, W4,
           g1, b1, gres, bres, g2, b2, g2d1, b2d1, g2d2, b2d2, g3, b3,
           g4, b4, gmid, bmid, gout, bout):
    v = lambda a: a.reshape(1, -1).astype(_f32)
    kr4 = lambda w: jnp.kron(jnp.eye(4, dtype=_f32), w)
    f32m = jnp.tile(jnp.eye(CM, dtype=_f32), (4, 1))
    f64m = jnp.tile(jnp.eye(CO, dtype=_f32), (4, 1))
    pk = lambda a: a.reshape(2, NP, 128)
    un = lambda a: a.reshape(-1, CM)

    src3r, dst3r, src3, dst3 = _prep_edges(edge_index)
    src2r, dst2r, _, _ = _prep_edges(edge_index_2d)
    src3p = jnp.concatenate([src3, src3 + N_PAD]).reshape(-1, CHUNK)
    dst3p = jnp.concatenate([dst3, dst3]).reshape(-1, CHUNK)

    xp = x.reshape(NR, 256)

    u = _k0(xp, kr4(W1), kr4(Wres))
    p = pk(_sc_apply(un(u), src3p, dst3p))
    st = _stats_pair(p, f32m)
    u2, r = _k2(p, st, v(g1), v(b1), v(gres), v(bres), kr4(W2))

    q = pk(_sc_apply(un(u2), src3r, dst3r, cpt0=CPT0))
    d3s, u2d1 = _k4(q, _stats_sum(q, f32m), v(g2), v(b2), r, kr4(W2d1))

    r3 = pk(_sc_apply(un(u2d1), src2r, dst2r, cpt0=CPT0))
    u2d2 = _k6(r3, _stats_sum(r3, f32m), v(g2d1), v(b2d1), kr4(W2d2))

    r4 = pk(_sc_apply(un(u2d2), src2r, dst2r, cpt0=CPT0))
    z, zs, zq = _k8(r4, _stats_sum(r4, f32m), v(g2d2), v(b2d2), d3s, f32m)

    u3 = _k9(z, (zs, zq), v(gmid), v(bmid), kr4(W3))

    r5 = pk(_sc_apply(un(u3), src3r, dst3r, cpt0=CPT0))
    e = _k11(r5, _stats_sum(r5, f32m), v(g3), v(b3))

    r6 = pk(_sc_apply(un(e), src3r, dst3r, cpt0=CPT0))
    a4, a4s, a4q = _k12(r6, kr4(W4), f64m)
    y, ys, yq = _k13(a4, (a4s, a4q), v(g4), v(b4), xp, f64m)
    return _k14(y, (ys, yq), v(gout), v(bout)).reshape(N, CO)
